# baseline (speedup 1.0000x reference)
import functools

import jax
import jax.numpy as jnp
from jax import lax
from jax.experimental import pallas as pl
from jax.experimental.pallas import tpu as pltpu
from jax.experimental.pallas import tpu_sc as plsc

N = 50000
E = 800000
IN_DIM = 1433
HID = 64
OUT_DIM = 7

NC = 2
NS = 16
NW = NC * NS
BN = 2176
NP = 23 * BN
GRID_N = NP // BN
N_TILE = NP // NS

CK = 4000
NCK = E // CK
NG = CK // 16
ECQ = E // 4

_mesh = plsc.VectorSubcoreMesh(core_axis_name="c", subcore_axis_name="s")
_sc_params = pltpu.CompilerParams(use_tc_tiling_on_sc=False, needs_layout_passes=False)


EPT = E // NW
NGD = EPT // 16


@functools.partial(
    pl.kernel,
    out_type=jax.ShapeDtypeStruct((NW, NP), jnp.float32),
    scratch_types=[
        pltpu.VMEM((NP,), jnp.float32),
        pltpu.VMEM((EPT + 24,), jnp.int32),
    ],
    mesh=_mesh,
    compiler_params=_sc_params,
)
def _sc_deg(dst_hbm, zeros_hbm, out_hbm, col_acc, dst_ch):
    c = lax.axis_index("c")
    s = lax.axis_index("s")
    wid = s * NC + c
    pltpu.sync_copy(zeros_hbm, col_acc)
    pltpu.sync_copy(dst_hbm.at[pl.ds(wid * EPT, EPT)],
                    dst_ch.at[pl.ds(0, EPT)])
    ones16 = jnp.ones((16,), jnp.float32)

    def grp(g, carry):
        dv = dst_ch[pl.ds(g * 16, 16)]
        plsc.addupdate_scatter(col_acc, [dv], ones16)
        return carry

    lax.fori_loop(0, NGD, grp, 0, unroll=8)
    tail = EPT - NGD * 16
    if tail:
        dv = dst_ch[pl.ds(NGD * 16, 16)]
        mask = lax.iota(jnp.int32, 16) < tail
        plsc.addupdate_scatter(col_acc, [dv], ones16, mask=mask)
    pltpu.sync_copy(col_acc, out_hbm.at[wid])


def _col_sweep(col_in, col_acc, src_hbm, dst_hbm, bufs, sems,
               e_base, n_chunks):
    s0, d0, s1, d1 = bufs
    sem0, sem1 = sems

    def start(ch, sb, db, sem):
        off = e_base + ch * CK
        pltpu.async_copy(src_hbm.at[pl.ds(off, CK)], sb, sem)
        pltpu.async_copy(dst_hbm.at[pl.ds(off, CK)], db, sem)

    def drain(sb, db, sem):
        pltpu.make_async_copy(src_hbm.at[pl.ds(0, CK)], sb, sem).wait()
        pltpu.make_async_copy(dst_hbm.at[pl.ds(0, CK)], db, sem).wait()

    def compute(sb, db):
        def grp(g, c2):
            b = g * 16
            sv = sb[pl.ds(b, 16)]
            dv = db[pl.ds(b, 16)]
            vals = plsc.load_gather(col_in, [sv])
            plsc.addupdate_scatter(col_acc, [dv], vals)
            return c2

        lax.fori_loop(0, NG, grp, 0, unroll=8)

    start(0, s0, d0, sem0)
    n_half = n_chunks // 2

    def body(i, carry):
        ch = 2 * i
        start(ch + 1, s1, d1, sem1)
        drain(s0, d0, sem0)
        compute(s0, d0)

        @pl.when(i < n_half - 1)
        def _():
            start(ch + 2, s0, d0, sem0)

        drain(s1, d1, sem1)
        compute(s1, d1)
        return carry

    lax.fori_loop(0, n_half, body, 0)


@functools.partial(
    pl.kernel,
    out_type=jax.ShapeDtypeStruct((HID, NP), jnp.float32),
    scratch_types=[
        pltpu.VMEM((NP,), jnp.float32),
        pltpu.VMEM((NP,), jnp.float32),
        pltpu.VMEM((CK,), jnp.int32),
        pltpu.VMEM((CK,), jnp.int32),
        pltpu.VMEM((CK,), jnp.int32),
        pltpu.VMEM((CK,), jnp.int32),
        pltpu.SemaphoreType.DMA,
        pltpu.SemaphoreType.DMA,
    ],
    mesh=_mesh,
    compiler_params=_sc_params,
)
def _sc_l1(src_hbm, dst_hbm, yt_hbm, out_hbm,
           col_in, col_acc, s0, d0, s1, d1, sem0, sem1):
    c = lax.axis_index("c")
    s = lax.axis_index("s")
    wid = s * NC + c
    for p in range(2):
        col = wid + 32 * p
        pltpu.sync_copy(yt_hbm.at[col], col_in)
        pltpu.sync_copy(yt_hbm.at[col], col_acc)
        _col_sweep(col_in, col_acc, src_hbm, dst_hbm,
                   (s0, d0, s1, d1), (sem0, sem1), 0, NCK)
        pltpu.sync_copy(col_acc, out_hbm.at[col])


@functools.partial(
    pl.kernel,
    out_type=jax.ShapeDtypeStruct((NW, NP), jnp.float32),
    scratch_types=[
        pltpu.VMEM((NP,), jnp.float32),
        pltpu.VMEM((NP,), jnp.float32),
        pltpu.VMEM((CK,), jnp.int32),
        pltpu.VMEM((CK,), jnp.int32),
        pltpu.VMEM((CK,), jnp.int32),
        pltpu.VMEM((CK,), jnp.int32),
        pltpu.SemaphoreType.DMA,
        pltpu.SemaphoreType.DMA,
    ],
    mesh=_mesh,
    compiler_params=_sc_params,
)
def _sc_l2(src_hbm, dst_hbm, y2t_hbm, zeros_hbm, out_hbm,
           col_in, col_acc, s0, d0, s1, d1, sem0, sem1):
    c = lax.axis_index("c")
    s = lax.axis_index("s")
    wid = s * NC + c
    col = lax.rem(wid, 8)
    q = lax.div(wid, 8)
    pltpu.sync_copy(y2t_hbm.at[col], col_in)

    @pl.when(q == 0)
    def _():
        pltpu.sync_copy(y2t_hbm.at[col], col_acc)

    @pl.when(q > 0)
    def _():
        pltpu.sync_copy(zeros_hbm, col_acc)

    _col_sweep(col_in, col_acc, src_hbm, dst_hbm,
               (s0, d0, s1, d1), (sem0, sem1), q * ECQ, ECQ // CK)
    pltpu.sync_copy(col_acc, out_hbm.at[wid])


def _m0_body(p_ref, d_ref):
    d_ref[...] = 1.0 + jnp.sum(p_ref[...], axis=0, keepdims=True)


def _m1_body(x_ref, w_ref, d_ref, yt_ref):
    dis = lax.rsqrt(d_ref[...])
    xwt = lax.dot_general(w_ref[...], x_ref[...],
                          (((0,), (1,)), ((), ())),
                          preferred_element_type=jnp.float32)
    yt_ref[...] = dis * xwt


def _m2_body(a_ref, d_ref, w2t_ref, b1_ref, y2t_ref):
    dis = lax.rsqrt(d_ref[...])
    h = jnp.maximum(dis * a_ref[...] + b1_ref[...], 0.0)
    y2t = lax.dot_general(w2t_ref[...], h,
                          (((1,), (0,)), ((), ())),
                          preferred_element_type=jnp.float32)
    y2t_ref[...] = dis * y2t


def _m3_body(p_ref, d_ref, b2_ref, o_ref):
    dis = lax.rsqrt(d_ref[...])
    p = p_ref[...]
    tot = p[0:8] + p[8:16] + p[16:24] + p[24:32]
    o_ref[...] = dis * tot + b2_ref[...]


def kernel(x, edge_index, W1, b1, W2, b2):
    ei = edge_index.astype(jnp.int32)
    src = ei[0]
    dst = ei[1]
    zeros_n = jnp.zeros((NP,), jnp.float32)

    dparts = _sc_deg(dst, zeros_n)
    d2 = pl.pallas_call(
        _m0_body,
        grid=(GRID_N,),
        in_specs=[pl.BlockSpec((NW, BN), lambda i: (0, i))],
        out_specs=pl.BlockSpec((1, BN), lambda i: (0, i)),
        out_shape=jax.ShapeDtypeStruct((1, NP), jnp.float32),
    )(dparts)

    yt = pl.pallas_call(
        _m1_body,
        grid=(GRID_N,),
        in_specs=[
            pl.BlockSpec((BN, IN_DIM), lambda i: (i, 0)),
            pl.BlockSpec((IN_DIM, HID), lambda i: (0, 0)),
            pl.BlockSpec((1, BN), lambda i: (0, i)),
        ],
        out_specs=pl.BlockSpec((HID, BN), lambda i: (0, i)),
        out_shape=jax.ShapeDtypeStruct((HID, NP), jnp.float32),
    )(x, W1, d2)

    at = _sc_l1(src, dst, yt)

    w2t = jnp.zeros((HID, 16), jnp.float32).at[:, :OUT_DIM].set(W2).T
    y2t = pl.pallas_call(
        _m2_body,
        grid=(GRID_N,),
        in_specs=[
            pl.BlockSpec((HID, BN), lambda i: (0, i)),
            pl.BlockSpec((1, BN), lambda i: (0, i)),
            pl.BlockSpec((16, HID), lambda i: (0, 0)),
            pl.BlockSpec((HID, 1), lambda i: (0, 0)),
        ],
        out_specs=pl.BlockSpec((16, BN), lambda i: (0, i)),
        out_shape=jax.ShapeDtypeStruct((16, NP), jnp.float32),
    )(at, d2, w2t, b1.reshape(HID, 1))

    pt = _sc_l2(src, dst, y2t, zeros_n)

    b2p = jnp.zeros((8, 1), jnp.float32).at[:OUT_DIM, 0].set(b2)
    outt = pl.pallas_call(
        _m3_body,
        grid=(GRID_N,),
        in_specs=[
            pl.BlockSpec((NW, BN), lambda i: (0, i)),
            pl.BlockSpec((1, BN), lambda i: (0, i)),
            pl.BlockSpec((8, 1), lambda i: (0, 0)),
        ],
        out_specs=pl.BlockSpec((8, BN), lambda i: (0, i)),
        out_shape=jax.ShapeDtypeStruct((8, NP), jnp.float32),
    )(pt, d2, b2p)
    return outt[:OUT_DIM, :N].T


# speedup vs baseline: 1.6958x; 1.6958x over previous
import functools

import jax
import jax.numpy as jnp
from jax import lax
from jax.experimental import pallas as pl
from jax.experimental.pallas import tpu as pltpu
from jax.experimental.pallas import tpu_sc as plsc

N = 50000
E = 800000
IN_DIM = 1433
HID = 64
OUT_DIM = 7

NC = 2
NS = 16
NW = NC * NS
BN = 2176
NP = 23 * BN
GRID_N = NP // BN
N_TILE = NP // NS

CK = 4000
NCK = E // CK
NG = CK // 16
KB = 5
ECQ = E // 4

_mesh = plsc.VectorSubcoreMesh(core_axis_name="c", subcore_axis_name="s")
_sc_params = pltpu.CompilerParams(use_tc_tiling_on_sc=False, needs_layout_passes=False)


EPT = E // NW
NGD = EPT // 16


@functools.partial(
    pl.kernel,
    out_type=jax.ShapeDtypeStruct((NW, NP), jnp.float32),
    scratch_types=[
        pltpu.VMEM((NP,), jnp.float32),
        pltpu.VMEM((EPT + 24,), jnp.int32),
    ],
    mesh=_mesh,
    compiler_params=_sc_params,
)
def _sc_deg(dst_hbm, zeros_hbm, out_hbm, col_acc, dst_ch):
    c = lax.axis_index("c")
    s = lax.axis_index("s")
    wid = s * NC + c
    pltpu.sync_copy(zeros_hbm, col_acc)
    pltpu.sync_copy(dst_hbm.at[pl.ds(wid * EPT, EPT)],
                    dst_ch.at[pl.ds(0, EPT)])
    ones16 = jnp.ones((16,), jnp.float32)

    def grp(g, carry):
        dv = dst_ch[pl.ds(g * 16, 16)]
        plsc.addupdate_scatter(col_acc, [dv], ones16)
        return carry

    lax.fori_loop(0, NGD, grp, 0, unroll=8)
    tail = EPT - NGD * 16
    if tail:
        dv = dst_ch[pl.ds(NGD * 16, 16)]
        mask = lax.iota(jnp.int32, 16) < tail
        plsc.addupdate_scatter(col_acc, [dv], ones16, mask=mask)
    pltpu.sync_copy(col_acc, out_hbm.at[wid])


def _col_sweep(col_in, col_acc, src_hbm, dst_hbm, bufs, sems,
               e_base, n_chunks):
    s0, d0, s1, d1 = bufs
    sem0, sem1 = sems

    def start(ch, sb, db, sem):
        off = e_base + ch * CK
        pltpu.async_copy(src_hbm.at[pl.ds(off, CK)], sb, sem)
        pltpu.async_copy(dst_hbm.at[pl.ds(off, CK)], db, sem)

    def drain(sb, db, sem):
        pltpu.make_async_copy(src_hbm.at[pl.ds(0, CK)], sb, sem).wait()
        pltpu.make_async_copy(dst_hbm.at[pl.ds(0, CK)], db, sem).wait()

    def compute(sb, db):
        def grp(g, c2):
            b = g * (16 * KB)
            svs = [sb[pl.ds(b + k * 16, 16)] for k in range(KB)]
            dvs = [db[pl.ds(b + k * 16, 16)] for k in range(KB)]
            vals = [plsc.load_gather(col_in, [sv]) for sv in svs]
            for k in range(KB):
                plsc.addupdate_scatter(col_acc, [dvs[k]], vals[k])
            return c2

        lax.fori_loop(0, NG // KB, grp, 0, unroll=2)

    start(0, s0, d0, sem0)
    n_half = n_chunks // 2

    def body(i, carry):
        ch = 2 * i
        start(ch + 1, s1, d1, sem1)
        drain(s0, d0, sem0)
        compute(s0, d0)

        @pl.when(i < n_half - 1)
        def _():
            start(ch + 2, s0, d0, sem0)

        drain(s1, d1, sem1)
        compute(s1, d1)
        return carry

    lax.fori_loop(0, n_half, body, 0)


@functools.partial(
    pl.kernel,
    out_type=jax.ShapeDtypeStruct((HID, NP), jnp.float32),
    scratch_types=[
        pltpu.VMEM((NP,), jnp.float32),
        pltpu.VMEM((NP,), jnp.float32),
        pltpu.VMEM((CK,), jnp.int32),
        pltpu.VMEM((CK,), jnp.int32),
        pltpu.VMEM((CK,), jnp.int32),
        pltpu.VMEM((CK,), jnp.int32),
        pltpu.SemaphoreType.DMA,
        pltpu.SemaphoreType.DMA,
    ],
    mesh=_mesh,
    compiler_params=_sc_params,
)
def _sc_l1(src_hbm, dst_hbm, yt_hbm, out_hbm,
           col_in, col_acc, s0, d0, s1, d1, sem0, sem1):
    c = lax.axis_index("c")
    s = lax.axis_index("s")
    wid = s * NC + c
    for p in range(2):
        col = wid + 32 * p
        pltpu.sync_copy(yt_hbm.at[col], col_in)
        pltpu.sync_copy(yt_hbm.at[col], col_acc)
        _col_sweep(col_in, col_acc, src_hbm, dst_hbm,
                   (s0, d0, s1, d1), (sem0, sem1), 0, NCK)
        pltpu.sync_copy(col_acc, out_hbm.at[col])


@functools.partial(
    pl.kernel,
    out_type=jax.ShapeDtypeStruct((NW, NP), jnp.float32),
    scratch_types=[
        pltpu.VMEM((NP,), jnp.float32),
        pltpu.VMEM((NP,), jnp.float32),
        pltpu.VMEM((CK,), jnp.int32),
        pltpu.VMEM((CK,), jnp.int32),
        pltpu.VMEM((CK,), jnp.int32),
        pltpu.VMEM((CK,), jnp.int32),
        pltpu.SemaphoreType.DMA,
        pltpu.SemaphoreType.DMA,
    ],
    mesh=_mesh,
    compiler_params=_sc_params,
)
def _sc_l2(src_hbm, dst_hbm, y2t_hbm, zeros_hbm, out_hbm,
           col_in, col_acc, s0, d0, s1, d1, sem0, sem1):
    c = lax.axis_index("c")
    s = lax.axis_index("s")
    wid = s * NC + c
    col = lax.rem(wid, 8)
    q = lax.div(wid, 8)
    pltpu.sync_copy(y2t_hbm.at[col], col_in)

    @pl.when(q == 0)
    def _():
        pltpu.sync_copy(y2t_hbm.at[col], col_acc)

    @pl.when(q > 0)
    def _():
        pltpu.sync_copy(zeros_hbm, col_acc)

    _col_sweep(col_in, col_acc, src_hbm, dst_hbm,
               (s0, d0, s1, d1), (sem0, sem1), q * ECQ, ECQ // CK)
    pltpu.sync_copy(col_acc, out_hbm.at[wid])


def _m0_body(p_ref, d_ref):
    d_ref[...] = 1.0 + jnp.sum(p_ref[...], axis=0, keepdims=True)


def _m1_body(x_ref, w_ref, d_ref, yt_ref):
    dis = lax.rsqrt(d_ref[...])
    xwt = lax.dot_general(w_ref[...], x_ref[...],
                          (((0,), (1,)), ((), ())),
                          preferred_element_type=jnp.float32)
    yt_ref[...] = dis * xwt


def _m2_body(a_ref, d_ref, w2t_ref, b1_ref, y2t_ref):
    dis = lax.rsqrt(d_ref[...])
    h = jnp.maximum(dis * a_ref[...] + b1_ref[...], 0.0)
    y2t = lax.dot_general(w2t_ref[...], h,
                          (((1,), (0,)), ((), ())),
                          preferred_element_type=jnp.float32)
    y2t_ref[...] = dis * y2t


def _m3_body(p_ref, d_ref, b2_ref, o_ref):
    dis = lax.rsqrt(d_ref[...])
    p = p_ref[...]
    tot = p[0:8] + p[8:16] + p[16:24] + p[24:32]
    o_ref[...] = dis * tot + b2_ref[...]


def kernel(x, edge_index, W1, b1, W2, b2):
    ei = edge_index.astype(jnp.int32)
    src = ei[0]
    dst = ei[1]
    zeros_n = jnp.zeros((NP,), jnp.float32)

    dparts = _sc_deg(dst, zeros_n)
    d2 = pl.pallas_call(
        _m0_body,
        grid=(GRID_N,),
        in_specs=[pl.BlockSpec((NW, BN), lambda i: (0, i))],
        out_specs=pl.BlockSpec((1, BN), lambda i: (0, i)),
        out_shape=jax.ShapeDtypeStruct((1, NP), jnp.float32),
    )(dparts)

    yt = pl.pallas_call(
        _m1_body,
        grid=(GRID_N,),
        in_specs=[
            pl.BlockSpec((BN, IN_DIM), lambda i: (i, 0)),
            pl.BlockSpec((IN_DIM, HID), lambda i: (0, 0)),
            pl.BlockSpec((1, BN), lambda i: (0, i)),
        ],
        out_specs=pl.BlockSpec((HID, BN), lambda i: (0, i)),
        out_shape=jax.ShapeDtypeStruct((HID, NP), jnp.float32),
    )(x, W1, d2)

    at = _sc_l1(src, dst, yt)

    w2t = jnp.zeros((HID, 16), jnp.float32).at[:, :OUT_DIM].set(W2).T
    y2t = pl.pallas_call(
        _m2_body,
        grid=(GRID_N,),
        in_specs=[
            pl.BlockSpec((HID, BN), lambda i: (0, i)),
            pl.BlockSpec((1, BN), lambda i: (0, i)),
            pl.BlockSpec((16, HID), lambda i: (0, 0)),
            pl.BlockSpec((HID, 1), lambda i: (0, 0)),
        ],
        out_specs=pl.BlockSpec((16, BN), lambda i: (0, i)),
        out_shape=jax.ShapeDtypeStruct((16, NP), jnp.float32),
    )(at, d2, w2t, b1.reshape(HID, 1))

    pt = _sc_l2(src, dst, y2t, zeros_n)

    b2p = jnp.zeros((8, 1), jnp.float32).at[:OUT_DIM, 0].set(b2)
    outt = pl.pallas_call(
        _m3_body,
        grid=(GRID_N,),
        in_specs=[
            pl.BlockSpec((NW, BN), lambda i: (0, i)),
            pl.BlockSpec((1, BN), lambda i: (0, i)),
            pl.BlockSpec((8, 1), lambda i: (0, 0)),
        ],
        out_specs=pl.BlockSpec((8, BN), lambda i: (0, i)),
        out_shape=jax.ShapeDtypeStruct((8, NP), jnp.float32),
    )(pt, d2, b2p)
    return outt[:OUT_DIM, :N].T


# speedup vs baseline: 1.7022x; 1.0038x over previous
import functools

import jax
import jax.numpy as jnp
from jax import lax
from jax.experimental import pallas as pl
from jax.experimental.pallas import tpu as pltpu
from jax.experimental.pallas import tpu_sc as plsc

N = 50000
E = 800000
IN_DIM = 1433
HID = 64
OUT_DIM = 7

NC = 2
NS = 16
NW = NC * NS
BN = 2176
NP = 23 * BN
GRID_N = NP // BN
N_TILE = NP // NS

CK = 4000
NCK = E // CK
NG = CK // 16
KB = 10
ECQ = E // 4

_mesh = plsc.VectorSubcoreMesh(core_axis_name="c", subcore_axis_name="s")
_sc_params = pltpu.CompilerParams(use_tc_tiling_on_sc=False, needs_layout_passes=False)


EPT = E // NW
NGD = EPT // 16


@functools.partial(
    pl.kernel,
    out_type=jax.ShapeDtypeStruct((NW, NP), jnp.float32),
    scratch_types=[
        pltpu.VMEM((NP,), jnp.float32),
        pltpu.VMEM((EPT + 24,), jnp.int32),
    ],
    mesh=_mesh,
    compiler_params=_sc_params,
)
def _sc_deg(dst_hbm, zeros_hbm, out_hbm, col_acc, dst_ch):
    c = lax.axis_index("c")
    s = lax.axis_index("s")
    wid = s * NC + c
    pltpu.sync_copy(zeros_hbm, col_acc)
    pltpu.sync_copy(dst_hbm.at[pl.ds(wid * EPT, EPT)],
                    dst_ch.at[pl.ds(0, EPT)])
    ones16 = jnp.ones((16,), jnp.float32)

    def grp(g, carry):
        dv = dst_ch[pl.ds(g * 16, 16)]
        plsc.addupdate_scatter(col_acc, [dv], ones16)
        return carry

    lax.fori_loop(0, NGD, grp, 0, unroll=8)
    tail = EPT - NGD * 16
    if tail:
        dv = dst_ch[pl.ds(NGD * 16, 16)]
        mask = lax.iota(jnp.int32, 16) < tail
        plsc.addupdate_scatter(col_acc, [dv], ones16, mask=mask)
    pltpu.sync_copy(col_acc, out_hbm.at[wid])


def _col_sweep(col_in, col_acc, src_hbm, dst_hbm, bufs, sems,
               e_base, n_chunks):
    s0, d0, s1, d1 = bufs
    sem0, sem1 = sems

    def start(ch, sb, db, sem):
        off = e_base + ch * CK
        pltpu.async_copy(src_hbm.at[pl.ds(off, CK)], sb, sem)
        pltpu.async_copy(dst_hbm.at[pl.ds(off, CK)], db, sem)

    def drain(sb, db, sem):
        pltpu.make_async_copy(src_hbm.at[pl.ds(0, CK)], sb, sem).wait()
        pltpu.make_async_copy(dst_hbm.at[pl.ds(0, CK)], db, sem).wait()

    def compute(sb, db):
        def grp(g, c2):
            b = g * (16 * KB)
            svs = [sb[pl.ds(b + k * 16, 16)] for k in range(KB)]
            dvs = [db[pl.ds(b + k * 16, 16)] for k in range(KB)]
            vals = [plsc.load_gather(col_in, [sv]) for sv in svs]
            for k in range(KB):
                plsc.addupdate_scatter(col_acc, [dvs[k]], vals[k])
            return c2

        lax.fori_loop(0, NG // KB, grp, 0)

    start(0, s0, d0, sem0)
    n_half = n_chunks // 2

    def body(i, carry):
        ch = 2 * i
        start(ch + 1, s1, d1, sem1)
        drain(s0, d0, sem0)
        compute(s0, d0)

        @pl.when(i < n_half - 1)
        def _():
            start(ch + 2, s0, d0, sem0)

        drain(s1, d1, sem1)
        compute(s1, d1)
        return carry

    lax.fori_loop(0, n_half, body, 0)


@functools.partial(
    pl.kernel,
    out_type=jax.ShapeDtypeStruct((HID, NP), jnp.float32),
    scratch_types=[
        pltpu.VMEM((NP,), jnp.float32),
        pltpu.VMEM((NP,), jnp.float32),
        pltpu.VMEM((CK,), jnp.int32),
        pltpu.VMEM((CK,), jnp.int32),
        pltpu.VMEM((CK,), jnp.int32),
        pltpu.VMEM((CK,), jnp.int32),
        pltpu.SemaphoreType.DMA,
        pltpu.SemaphoreType.DMA,
    ],
    mesh=_mesh,
    compiler_params=_sc_params,
)
def _sc_l1(src_hbm, dst_hbm, yt_hbm, out_hbm,
           col_in, col_acc, s0, d0, s1, d1, sem0, sem1):
    c = lax.axis_index("c")
    s = lax.axis_index("s")
    wid = s * NC + c
    for p in range(2):
        col = wid + 32 * p
        pltpu.sync_copy(yt_hbm.at[col], col_in)
        pltpu.sync_copy(yt_hbm.at[col], col_acc)
        _col_sweep(col_in, col_acc, src_hbm, dst_hbm,
                   (s0, d0, s1, d1), (sem0, sem1), 0, NCK)
        pltpu.sync_copy(col_acc, out_hbm.at[col])


@functools.partial(
    pl.kernel,
    out_type=jax.ShapeDtypeStruct((NW, NP), jnp.float32),
    scratch_types=[
        pltpu.VMEM((NP,), jnp.float32),
        pltpu.VMEM((NP,), jnp.float32),
        pltpu.VMEM((CK,), jnp.int32),
        pltpu.VMEM((CK,), jnp.int32),
        pltpu.VMEM((CK,), jnp.int32),
        pltpu.VMEM((CK,), jnp.int32),
        pltpu.SemaphoreType.DMA,
        pltpu.SemaphoreType.DMA,
    ],
    mesh=_mesh,
    compiler_params=_sc_params,
)
def _sc_l2(src_hbm, dst_hbm, y2t_hbm, zeros_hbm, out_hbm,
           col_in, col_acc, s0, d0, s1, d1, sem0, sem1):
    c = lax.axis_index("c")
    s = lax.axis_index("s")
    wid = s * NC + c
    col = lax.rem(wid, 8)
    q = lax.div(wid, 8)
    pltpu.sync_copy(y2t_hbm.at[col], col_in)

    @pl.when(q == 0)
    def _():
        pltpu.sync_copy(y2t_hbm.at[col], col_acc)

    @pl.when(q > 0)
    def _():
        pltpu.sync_copy(zeros_hbm, col_acc)

    _col_sweep(col_in, col_acc, src_hbm, dst_hbm,
               (s0, d0, s1, d1), (sem0, sem1), q * ECQ, ECQ // CK)
    pltpu.sync_copy(col_acc, out_hbm.at[wid])


def _m0_body(p_ref, d_ref):
    d_ref[...] = 1.0 + jnp.sum(p_ref[...], axis=0, keepdims=True)


def _m1_body(x_ref, w_ref, d_ref, yt_ref):
    dis = lax.rsqrt(d_ref[...])
    xwt = lax.dot_general(w_ref[...], x_ref[...],
                          (((0,), (1,)), ((), ())),
                          preferred_element_type=jnp.float32)
    yt_ref[...] = dis * xwt


def _m2_body(a_ref, d_ref, w2t_ref, b1_ref, y2t_ref):
    dis = lax.rsqrt(d_ref[...])
    h = jnp.maximum(dis * a_ref[...] + b1_ref[...], 0.0)
    y2t = lax.dot_general(w2t_ref[...], h,
                          (((1,), (0,)), ((), ())),
                          preferred_element_type=jnp.float32)
    y2t_ref[...] = dis * y2t


def _m3_body(p_ref, d_ref, b2_ref, o_ref):
    dis = lax.rsqrt(d_ref[...])
    p = p_ref[...]
    tot = p[0:8] + p[8:16] + p[16:24] + p[24:32]
    o_ref[...] = dis * tot + b2_ref[...]


def kernel(x, edge_index, W1, b1, W2, b2):
    ei = edge_index.astype(jnp.int32)
    src = ei[0]
    dst = ei[1]
    zeros_n = jnp.zeros((NP,), jnp.float32)

    dparts = _sc_deg(dst, zeros_n)
    d2 = pl.pallas_call(
        _m0_body,
        grid=(GRID_N,),
        in_specs=[pl.BlockSpec((NW, BN), lambda i: (0, i))],
        out_specs=pl.BlockSpec((1, BN), lambda i: (0, i)),
        out_shape=jax.ShapeDtypeStruct((1, NP), jnp.float32),
    )(dparts)

    yt = pl.pallas_call(
        _m1_body,
        grid=(GRID_N,),
        in_specs=[
            pl.BlockSpec((BN, IN_DIM), lambda i: (i, 0)),
            pl.BlockSpec((IN_DIM, HID), lambda i: (0, 0)),
            pl.BlockSpec((1, BN), lambda i: (0, i)),
        ],
        out_specs=pl.BlockSpec((HID, BN), lambda i: (0, i)),
        out_shape=jax.ShapeDtypeStruct((HID, NP), jnp.float32),
    )(x, W1, d2)

    at = _sc_l1(src, dst, yt)

    w2t = jnp.zeros((HID, 16), jnp.float32).at[:, :OUT_DIM].set(W2).T
    y2t = pl.pallas_call(
        _m2_body,
        grid=(GRID_N,),
        in_specs=[
            pl.BlockSpec((HID, BN), lambda i: (0, i)),
            pl.BlockSpec((1, BN), lambda i: (0, i)),
            pl.BlockSpec((16, HID), lambda i: (0, 0)),
            pl.BlockSpec((HID, 1), lambda i: (0, 0)),
        ],
        out_specs=pl.BlockSpec((16, BN), lambda i: (0, i)),
        out_shape=jax.ShapeDtypeStruct((16, NP), jnp.float32),
    )(at, d2, w2t, b1.reshape(HID, 1))

    pt = _sc_l2(src, dst, y2t, zeros_n)

    b2p = jnp.zeros((8, 1), jnp.float32).at[:OUT_DIM, 0].set(b2)
    outt = pl.pallas_call(
        _m3_body,
        grid=(GRID_N,),
        in_specs=[
            pl.BlockSpec((NW, BN), lambda i: (0, i)),
            pl.BlockSpec((1, BN), lambda i: (0, i)),
            pl.BlockSpec((8, 1), lambda i: (0, 0)),
        ],
        out_specs=pl.BlockSpec((8, BN), lambda i: (0, i)),
        out_shape=jax.ShapeDtypeStruct((8, NP), jnp.float32),
    )(pt, d2, b2p)
    return outt[:OUT_DIM, :N].T
